# baseline (speedup 1.0000x reference)
import functools

import jax
import jax.numpy as jnp
import numpy as np
from jax import lax
from jax.experimental import pallas as pl
from jax.experimental.pallas import tpu as pltpu
from jax.experimental.pallas import tpu_sc as plsc

N_NODES = 10000
N_EDGES = 160000
D_IN = 128
DIMS_IN = [128, 120, 120, 120]
DIMS_OUT = [144, 144, 144, 128]
N_BASIS = 10
N_SH = 9

CH = 64
NCH = 158
PER_TILE = CH * NCH
E_PAD = 16 * PER_TILE
SPLITW = 80
BLK_A = 1024
N_PAD = 10240
ROWS_PER_TILE = N_PAD // 16
OUT_BLK = 128

_P_GATE = np.zeros((24, 88), np.float32)
_col = 0
for _g in range(16):
    _P_GATE[_g, _col:_col + 3] = 1.0
    _col += 3
for _g in range(8):
    _P_GATE[16 + _g, _col:_col + 5] = 1.0
    _col += 5


def _coeff_body(ax_ref, ay_ref, az_ref, w0_ref, b0_ref, w1_ref, b1_ref,
                w2_ref, b2_ref, o0_ref, o1_ref, o2_ref, o3_ref):
    i = pl.program_id(0)
    vx = ax_ref[...]
    vy = ay_ref[...]
    vz = az_ref[...]
    r = jnp.sqrt(vx * vx + vy * vy + vz * vz + 1e-12)
    ux = vx / r
    uy = vy / r
    uz = vz / r
    zero = jnp.zeros_like(ux)
    ys = [
        0.28209479177 * jnp.ones_like(ux),
        0.4886025119 * uy,
        0.4886025119 * uz,
        0.4886025119 * ux,
        1.0925484306 * ux * uy,
        1.0925484306 * uy * uz,
        0.31539156525 * (3.0 * uz * uz - 1.0),
        1.0925484306 * ux * uz,
        0.54627421529 * (ux * ux - uy * uy),
    ]
    ymat = jnp.stack(ys + [zero] * 7, axis=0)
    bs = []
    for k in range(N_BASIS):
        mu = k / (N_BASIS - 1.0)
        bs.append(jnp.exp(-0.5 * ((r - mu) * N_BASIS) ** 2))
    bt = jnp.stack(bs + [zero] * 6, axis=0)
    eidx = i * BLK_A + lax.broadcasted_iota(jnp.int32, (16, BLK_A), 1)
    valid = jnp.where(eidx < N_EDGES, 1.0, 0.0).astype(jnp.float32)
    outs = [o0_ref, o1_ref, o2_ref, o3_ref]
    for L in range(4):
        h = jnp.dot(w0_ref[L], bt, preferred_element_type=jnp.float32)
        h = h + b0_ref[L]
        h = h * jax.nn.sigmoid(h)
        h = jnp.dot(w1_ref[L], h, preferred_element_type=jnp.float32)
        h = h + b1_ref[L]
        h = h * jax.nn.sigmoid(h)
        rr = jnp.dot(w2_ref[L], h, preferred_element_type=jnp.float32)
        rr = rr + b2_ref[L]
        ct = rr * ymat * valid
        outs[L][...] = ct.T


def _coeff_call(ax, ay, az, w0, b0, w1, b1, w2, b2):
    grid = E_PAD // BLK_A
    edge_spec = pl.BlockSpec((BLK_A,), lambda i: (i,))
    full = lambda a: pl.BlockSpec(a.shape, lambda i: tuple(0 for _ in a.shape))
    out_spec = pl.BlockSpec((BLK_A, 16), lambda i: (i, 0))
    return pl.pallas_call(
        _coeff_body,
        grid=(grid,),
        in_specs=[edge_spec, edge_spec, edge_spec,
                  full(w0), full(b0), full(w1), full(b1), full(w2), full(b2)],
        out_specs=[out_spec] * 4,
        out_shape=[jax.ShapeDtypeStruct((E_PAD, 16), jnp.float32)] * 4,
    )(ax, ay, az, w0, b0, w1, b1, w2, b2)


def _mm_body(v_ref, w_ref, o_ref):
    o_ref[0] = jnp.dot(v_ref[...], w_ref[0],
                       preferred_element_type=jnp.float32)


def _mm_call(v, wsplit):
    br = 400
    grid = N_NODES // br
    din = v.shape[1]
    nout = wsplit.shape[2]
    return pl.pallas_call(
        _mm_body,
        grid=(2, grid),
        in_specs=[pl.BlockSpec((br, din), lambda c, i: (i, 0)),
                  pl.BlockSpec((1, din, nout), lambda c, i: (c, 0, 0))],
        out_specs=pl.BlockSpec((1, br, nout), lambda c, i: (c, i, 0)),
        out_shape=jax.ShapeDtypeStruct((2, N_NODES, nout), jnp.float32),
    )(v, wsplit)


def _make_sc_conv(dout):
    nout = N_SH * SPLITW
    mesh = plsc.VectorSubcoreMesh(core_axis_name="c", subcore_axis_name="s")

    def body(t_hbm, cf_hbm, src_hbm, dst_hbm, out_hbm,
             acc_sh, idx_buf, dst_buf, cf_buf, g_buf, msg_buf, zb):
        c = lax.axis_index("c")
        s = lax.axis_index("s")

        def zrow(i, carry):
            for o in range(SPLITW // 16):
                zb[i, pl.ds(o * 16, 16)] = jnp.zeros((16,), jnp.float32)
            return carry
        lax.fori_loop(0, OUT_BLK, zrow, 0)
        for b in range(ROWS_PER_TILE // OUT_BLK):
            pltpu.sync_copy(
                zb, acc_sh.at[pl.ds(s * ROWS_PER_TILE + b * OUT_BLK, OUT_BLK)])
        plsc.subcore_barrier()

        ebase = s * PER_TILE

        def chunk(g, carry):
            base = ebase + g * CH
            pltpu.sync_copy(src_hbm.at[pl.ds(base, CH)], idx_buf)
            pltpu.sync_copy(dst_hbm.at[pl.ds(base, CH)], dst_buf)
            pltpu.sync_copy(cf_hbm.at[pl.ds(base, CH)], cf_buf)
            off = jnp.broadcast_to(c * N_NODES, (16,)).astype(jnp.int32)
            for k in range(CH // 16):
                idx_buf[pl.ds(k * 16, 16)] = idx_buf[pl.ds(k * 16, 16)] + off
            pltpu.sync_copy(t_hbm.at[idx_buf], g_buf)

            def edge(j, cc):
                crow = cf_buf[j]
                cs = [jnp.broadcast_to(crow[m], (16,)) for m in range(N_SH)]
                for o in range(SPLITW // 16):
                    acc = cs[0] * g_buf[j, pl.ds(o * 16, 16)]
                    for m in range(1, N_SH):
                        acc = acc + cs[m] * g_buf[j, pl.ds(m * SPLITW + o * 16, 16)]
                    msg_buf[j, pl.ds(o * 16, 16)] = acc
                return cc
            lax.fori_loop(0, CH, edge, 0)
            pltpu.sync_copy(msg_buf, acc_sh.at[dst_buf], add=True)
            return carry
        lax.fori_loop(0, NCH, chunk, 0)
        plsc.subcore_barrier()

        for b in range(ROWS_PER_TILE // OUT_BLK):
            row = s * ROWS_PER_TILE + b * OUT_BLK
            pltpu.sync_copy(acc_sh.at[pl.ds(row, OUT_BLK)], zb)
            pltpu.sync_copy(zb, out_hbm.at[c, pl.ds(row, OUT_BLK)])

    return pl.kernel(
        body,
        mesh=mesh,
        compiler_params=pltpu.CompilerParams(use_tc_tiling_on_sc=False),
        out_type=jax.ShapeDtypeStruct((2, N_PAD, SPLITW), jnp.float32),
        scratch_types=[
            pltpu.VMEM_SHARED((N_PAD, SPLITW), jnp.float32),
            pltpu.VMEM((CH,), jnp.int32),
            pltpu.VMEM((CH,), jnp.int32),
            pltpu.VMEM((CH, 16), jnp.float32),
            pltpu.VMEM((CH, nout), jnp.float32),
            pltpu.VMEM((CH, SPLITW), jnp.float32),
            pltpu.VMEM((OUT_BLK, SPLITW), jnp.float32),
        ],
    )


_SC_CONV = _make_sc_conv(144)


def _gate_body(p0_ref, p1_ref, pg_ref, o_ref):
    h = jnp.concatenate([p0_ref[...], p1_ref[...][:, :64]], axis=1) * 0.25
    sc = h[:, :32]
    s = sc * jax.nn.sigmoid(sc)
    gts = jax.nn.sigmoid(h[:, 32:56])
    ge = jnp.dot(gts, pg_ref[...], preferred_element_type=jnp.float32)
    o_ref[...] = jnp.concatenate([s, h[:, 56:144] * ge], axis=1)


def _gate_call(p0, p1, pg):
    br = 400
    grid = N_NODES // br
    return pl.pallas_call(
        _gate_body,
        grid=(grid,),
        in_specs=[pl.BlockSpec((br, SPLITW), lambda i: (i, 0)),
                  pl.BlockSpec((br, SPLITW), lambda i: (i, 0)),
                  pl.BlockSpec((24, 88), lambda i: (0, 0))],
        out_specs=pl.BlockSpec((br, 120), lambda i: (i, 0)),
        out_shape=jax.ShapeDtypeStruct((N_NODES, 120), jnp.float32),
    )(p0, p1, pg)


def _final_body(p0_ref, p1_ref, o_ref):
    o_ref[...] = jnp.concatenate(
        [p0_ref[...], p1_ref[...][:, :48]], axis=1) * 0.25


def _final_call(p0, p1):
    br = 400
    grid = N_NODES // br
    return pl.pallas_call(
        _final_body,
        grid=(grid,),
        in_specs=[pl.BlockSpec((br, SPLITW), lambda i: (i, 0)),
                  pl.BlockSpec((br, SPLITW), lambda i: (i, 0))],
        out_specs=pl.BlockSpec((br, 128), lambda i: (i, 0)),
        out_shape=jax.ShapeDtypeStruct((N_NODES, 128), jnp.float32),
    )(p0, p1)


def kernel(x, edge_index, edge_attr, params):
    pad = E_PAD - N_EDGES
    src = jnp.concatenate([edge_index[0], jnp.zeros((pad,), jnp.int32)])
    dst = jnp.concatenate([edge_index[1], jnp.zeros((pad,), jnp.int32)])
    eap = jnp.concatenate([edge_attr, jnp.zeros((pad, 3), jnp.float32)], axis=0)
    ax, ay, az = eap[:, 0], eap[:, 1], eap[:, 2]

    w0 = jnp.stack([
        jnp.pad(params["R%d_w0" % L].T, ((0, 0), (0, 6))) for L in range(4)])
    b0 = jnp.stack([params["R%d_b0" % L][:, None] for L in range(4)])
    w1 = jnp.stack([params["R%d_w1" % L].T for L in range(4)])
    b1 = jnp.stack([params["R%d_b1" % L][:, None] for L in range(4)])
    w2 = jnp.stack([
        jnp.pad(params["R%d_w2" % L].T, ((0, 7), (0, 0))) for L in range(4)])
    b2 = jnp.stack([
        jnp.pad(params["R%d_b2" % L], (0, 7))[:, None] for L in range(4)])
    coeffs = _coeff_call(ax, ay, az, w0, b0, w1, b1, w2, b2)

    pg = jnp.asarray(_P_GATE)
    v = x
    for L in range(4):
        dout = DIMS_OUT[L]
        din = DIMS_IN[L]
        wt = jnp.transpose(params["W%d" % L], (1, 0, 2))
        wc0 = wt[:, :, :SPLITW].reshape(din, N_SH * SPLITW)
        wc1 = jnp.pad(wt[:, :, SPLITW:],
                      ((0, 0), (0, 0), (0, 2 * SPLITW - dout))
                      ).reshape(din, N_SH * SPLITW)
        wsplit = jnp.stack([wc0, wc1])
        t = _mm_call(v, wsplit)
        t2 = t.reshape(2 * N_NODES, N_SH * SPLITW)
        part = _SC_CONV(t2, coeffs[L], src, dst)
        p0, p1 = part[0, :N_NODES], part[1, :N_NODES]
        if L < 3:
            v = _gate_call(p0, p1, pg)
        else:
            return _final_call(p0, p1)


# speedup vs baseline: 1.6419x; 1.6419x over previous
import functools

import jax
import jax.numpy as jnp
import numpy as np
from jax import lax
from jax.experimental import pallas as pl
from jax.experimental.pallas import tpu as pltpu
from jax.experimental.pallas import tpu_sc as plsc

N_NODES = 10000
N_EDGES = 160000
D_IN = 128
DIMS_IN = [128, 120, 120, 120]
DIMS_OUT = [144, 144, 144, 128]
N_BASIS = 10
N_SH = 9

CH = 32
NCHT = 316
PER_TILE = CH * NCHT
E_PAD = 16 * PER_TILE
SPLITW = 80
BLK_A = 1024
N_PAD = 10240
ROWS_PER_TILE = N_PAD // 16
OUT_BLK = 32

_P_GATE = np.zeros((24, 88), np.float32)
_col = 0
for _g in range(16):
    _P_GATE[_g, _col:_col + 3] = 1.0
    _col += 3
for _g in range(8):
    _P_GATE[16 + _g, _col:_col + 5] = 1.0
    _col += 5


def _coeff_body(ax_ref, ay_ref, az_ref, w0_ref, b0_ref, w1_ref, b1_ref,
                w2_ref, b2_ref, o0_ref, o1_ref, o2_ref, o3_ref):
    i = pl.program_id(0)
    vx = ax_ref[...]
    vy = ay_ref[...]
    vz = az_ref[...]
    r = jnp.sqrt(vx * vx + vy * vy + vz * vz + 1e-12)
    ux = vx / r
    uy = vy / r
    uz = vz / r
    zero = jnp.zeros_like(ux)
    ys = [
        0.28209479177 * jnp.ones_like(ux),
        0.4886025119 * uy,
        0.4886025119 * uz,
        0.4886025119 * ux,
        1.0925484306 * ux * uy,
        1.0925484306 * uy * uz,
        0.31539156525 * (3.0 * uz * uz - 1.0),
        1.0925484306 * ux * uz,
        0.54627421529 * (ux * ux - uy * uy),
    ]
    ymat = jnp.stack(ys + [zero] * 7, axis=0)
    bs = []
    for k in range(N_BASIS):
        mu = k / (N_BASIS - 1.0)
        bs.append(jnp.exp(-0.5 * ((r - mu) * N_BASIS) ** 2))
    bt = jnp.stack(bs + [zero] * 6, axis=0)
    eidx = i * BLK_A + lax.broadcasted_iota(jnp.int32, (16, BLK_A), 1)
    valid = jnp.where(eidx < N_EDGES, 1.0, 0.0).astype(jnp.float32)
    outs = [o0_ref, o1_ref, o2_ref, o3_ref]
    for L in range(4):
        h = jnp.dot(w0_ref[L], bt, preferred_element_type=jnp.float32)
        h = h + b0_ref[L]
        h = h * jax.nn.sigmoid(h)
        h = jnp.dot(w1_ref[L], h, preferred_element_type=jnp.float32)
        h = h + b1_ref[L]
        h = h * jax.nn.sigmoid(h)
        rr = jnp.dot(w2_ref[L], h, preferred_element_type=jnp.float32)
        rr = rr + b2_ref[L]
        ct = rr * ymat * valid
        outs[L][...] = ct.T


def _coeff_call(ax, ay, az, w0, b0, w1, b1, w2, b2):
    grid = E_PAD // BLK_A
    edge_spec = pl.BlockSpec((BLK_A,), lambda i: (i,))
    full = lambda a: pl.BlockSpec(a.shape, lambda i: tuple(0 for _ in a.shape))
    out_spec = pl.BlockSpec((BLK_A, 16), lambda i: (i, 0))
    return pl.pallas_call(
        _coeff_body,
        grid=(grid,),
        in_specs=[edge_spec, edge_spec, edge_spec,
                  full(w0), full(b0), full(w1), full(b1), full(w2), full(b2)],
        out_specs=[out_spec] * 4,
        out_shape=[jax.ShapeDtypeStruct((E_PAD, 16), jnp.float32)] * 4,
    )(ax, ay, az, w0, b0, w1, b1, w2, b2)


def _mm_body(v_ref, w_ref, o_ref):
    o_ref[0] = jnp.dot(v_ref[...], w_ref[0],
                       preferred_element_type=jnp.float32)


def _mm_call(v, wsplit):
    br = 400
    grid = N_NODES // br
    din = v.shape[1]
    nout = wsplit.shape[2]
    return pl.pallas_call(
        _mm_body,
        grid=(2, grid),
        in_specs=[pl.BlockSpec((br, din), lambda c, i: (i, 0)),
                  pl.BlockSpec((1, din, nout), lambda c, i: (c, 0, 0))],
        out_specs=pl.BlockSpec((1, br, nout), lambda c, i: (c, i, 0)),
        out_shape=jax.ShapeDtypeStruct((2, N_NODES, nout), jnp.float32),
    )(v, wsplit)


def _make_sc_conv(dout):
    nout = N_SH * SPLITW
    mesh = plsc.VectorSubcoreMesh(core_axis_name="c", subcore_axis_name="s")

    def body(t_hbm, cf_hbm, src_hbm, dst_hbm, out_hbm,
             acc_sh, src_all, dst_all, cf_buf, g_buf, msg_buf, zb,
             gsem, csem, ssem):
        c = lax.axis_index("c")
        s = lax.axis_index("s")

        pltpu.sync_copy(src_hbm.at[pl.ds(s * NCHT, NCHT)], src_all)
        pltpu.sync_copy(dst_hbm.at[pl.ds(s * NCHT, NCHT)], dst_all)
        off = jnp.broadcast_to(c * N_NODES, (16,)).astype(jnp.int32)

        def addoff(g, carry):
            for k in range(CH // 16):
                src_all[g, pl.ds(k * 16, 16)] = (
                    src_all[g, pl.ds(k * 16, 16)] + off)
            return carry
        lax.fori_loop(0, NCHT, addoff, 0)

        def zrow(i, carry):
            for o in range(SPLITW // 16):
                zb[i, pl.ds(o * 16, 16)] = jnp.zeros((16,), jnp.float32)
            return carry
        lax.fori_loop(0, OUT_BLK, zrow, 0)
        for b in range(ROWS_PER_TILE // OUT_BLK):
            pltpu.sync_copy(
                zb, acc_sh.at[pl.ds(s * ROWS_PER_TILE + b * OUT_BLK, OUT_BLK)])
        plsc.subcore_barrier()

        ebase = s * PER_TILE

        def issue(gg, b):
            pltpu.async_copy(cf_hbm.at[pl.ds(ebase + gg * CH, CH)],
                             cf_buf.at[b], csem.at[b])
            pltpu.async_copy(t_hbm.at[src_all.at[gg]], g_buf.at[b],
                             gsem.at[b])

        issue(0, 0)
        issue(1, 1)

        def loop(i, carry):
            for b in range(2):
                gg = i * 2 + b
                pltpu.make_async_copy(cf_hbm.at[pl.ds(0, CH)], cf_buf.at[b],
                                      csem.at[b]).wait()
                pltpu.make_async_copy(t_hbm.at[pl.ds(0, CH)], g_buf.at[b],
                                      gsem.at[b]).wait()

                @pl.when(i > 0)
                def _wait_scatter():
                    pltpu.make_async_copy(
                        msg_buf.at[b], acc_sh.at[dst_all.at[0]],
                        ssem.at[b]).wait()

                def edge(j, cc):
                    crow = cf_buf[b, j]
                    cs = [jnp.broadcast_to(crow[m], (16,))
                          for m in range(N_SH)]
                    for o in range(SPLITW // 16):
                        acc = cs[0] * g_buf[b, j, pl.ds(o * 16, 16)]
                        for m in range(1, N_SH):
                            acc = acc + cs[m] * g_buf[
                                b, j, pl.ds(m * SPLITW + o * 16, 16)]
                        msg_buf[b, j, pl.ds(o * 16, 16)] = acc
                    return cc
                lax.fori_loop(0, CH, edge, 0)

                pltpu.async_copy(msg_buf.at[b], acc_sh.at[dst_all.at[gg]],
                                 ssem.at[b], add=True)

                @pl.when(gg + 2 < NCHT)
                def _issue_next():
                    issue(gg + 2, b)
            return carry
        lax.fori_loop(0, NCHT // 2, loop, 0)
        for b in range(2):
            pltpu.make_async_copy(msg_buf.at[b], acc_sh.at[dst_all.at[0]],
                                  ssem.at[b]).wait()
        plsc.subcore_barrier()

        for b in range(ROWS_PER_TILE // OUT_BLK):
            row = s * ROWS_PER_TILE + b * OUT_BLK
            pltpu.sync_copy(acc_sh.at[pl.ds(row, OUT_BLK)], zb)
            pltpu.sync_copy(zb, out_hbm.at[c, pl.ds(row, OUT_BLK)])

    return pl.kernel(
        body,
        mesh=mesh,
        compiler_params=pltpu.CompilerParams(use_tc_tiling_on_sc=False),
        out_type=jax.ShapeDtypeStruct((2, N_PAD, SPLITW), jnp.float32),
        scratch_types=[
            pltpu.VMEM_SHARED((N_PAD, SPLITW), jnp.float32),
            pltpu.VMEM((NCHT, CH), jnp.int32),
            pltpu.VMEM((NCHT, CH), jnp.int32),
            pltpu.VMEM((2, CH, 16), jnp.float32),
            pltpu.VMEM((2, CH, nout), jnp.float32),
            pltpu.VMEM((2, CH, SPLITW), jnp.float32),
            pltpu.VMEM((OUT_BLK, SPLITW), jnp.float32),
            pltpu.SemaphoreType.DMA((2,)),
            pltpu.SemaphoreType.DMA((2,)),
            pltpu.SemaphoreType.DMA((2,)),
        ],
    )


_SC_CONV = _make_sc_conv(144)


def _gate_body(p0_ref, p1_ref, pg_ref, o_ref):
    h = jnp.concatenate([p0_ref[...], p1_ref[...][:, :64]], axis=1) * 0.25
    sc = h[:, :32]
    s = sc * jax.nn.sigmoid(sc)
    gts = jax.nn.sigmoid(h[:, 32:56])
    ge = jnp.dot(gts, pg_ref[...], preferred_element_type=jnp.float32)
    o_ref[...] = jnp.concatenate([s, h[:, 56:144] * ge], axis=1)


def _gate_call(p0, p1, pg):
    br = 400
    grid = N_NODES // br
    return pl.pallas_call(
        _gate_body,
        grid=(grid,),
        in_specs=[pl.BlockSpec((br, SPLITW), lambda i: (i, 0)),
                  pl.BlockSpec((br, SPLITW), lambda i: (i, 0)),
                  pl.BlockSpec((24, 88), lambda i: (0, 0))],
        out_specs=pl.BlockSpec((br, 120), lambda i: (i, 0)),
        out_shape=jax.ShapeDtypeStruct((N_NODES, 120), jnp.float32),
    )(p0, p1, pg)


def _final_body(p0_ref, p1_ref, o_ref):
    o_ref[...] = jnp.concatenate(
        [p0_ref[...], p1_ref[...][:, :48]], axis=1) * 0.25


def _final_call(p0, p1):
    br = 400
    grid = N_NODES // br
    return pl.pallas_call(
        _final_body,
        grid=(grid,),
        in_specs=[pl.BlockSpec((br, SPLITW), lambda i: (i, 0)),
                  pl.BlockSpec((br, SPLITW), lambda i: (i, 0))],
        out_specs=pl.BlockSpec((br, 128), lambda i: (i, 0)),
        out_shape=jax.ShapeDtypeStruct((N_NODES, 128), jnp.float32),
    )(p0, p1)


def kernel(x, edge_index, edge_attr, params):
    pad = E_PAD - N_EDGES
    src2 = jnp.concatenate(
        [edge_index[0], jnp.zeros((pad,), jnp.int32)]).reshape(-1, CH)
    dst2 = jnp.concatenate(
        [edge_index[1], jnp.zeros((pad,), jnp.int32)]).reshape(-1, CH)
    eap = jnp.concatenate([edge_attr, jnp.zeros((pad, 3), jnp.float32)], axis=0)
    ax, ay, az = eap[:, 0], eap[:, 1], eap[:, 2]

    w0 = jnp.stack([
        jnp.pad(params["R%d_w0" % L].T, ((0, 0), (0, 6))) for L in range(4)])
    b0 = jnp.stack([params["R%d_b0" % L][:, None] for L in range(4)])
    w1 = jnp.stack([params["R%d_w1" % L].T for L in range(4)])
    b1 = jnp.stack([params["R%d_b1" % L][:, None] for L in range(4)])
    w2 = jnp.stack([
        jnp.pad(params["R%d_w2" % L].T, ((0, 7), (0, 0))) for L in range(4)])
    b2 = jnp.stack([
        jnp.pad(params["R%d_b2" % L], (0, 7))[:, None] for L in range(4)])
    coeffs = _coeff_call(ax, ay, az, w0, b0, w1, b1, w2, b2)

    pg = jnp.asarray(_P_GATE)
    v = x
    for L in range(4):
        dout = DIMS_OUT[L]
        din = DIMS_IN[L]
        wt = jnp.transpose(params["W%d" % L], (1, 0, 2))
        wc0 = wt[:, :, :SPLITW].reshape(din, N_SH * SPLITW)
        wc1 = jnp.pad(wt[:, :, SPLITW:],
                      ((0, 0), (0, 0), (0, 2 * SPLITW - dout))
                      ).reshape(din, N_SH * SPLITW)
        wsplit = jnp.stack([wc0, wc1])
        t = _mm_call(v, wsplit)
        t2 = t.reshape(2 * N_NODES, N_SH * SPLITW)
        part = _SC_CONV(t2, coeffs[L], src2, dst2)
        p0, p1 = part[0, :N_NODES], part[1, :N_NODES]
        if L < 3:
            v = _gate_call(p0, p1, pg)
        else:
            return _final_call(p0, p1)
